# baseline (speedup 1.0000x reference)
import functools

import jax
import jax.numpy as jnp
from jax import lax
from jax.experimental import pallas as pl
from jax.experimental.pallas import tpu as pltpu
from jax.experimental.pallas import tpu_sc as plsc

BW = 128


GATHER_WINDOW = 512


def _sc_gather(table, ids1):
    n = ids1.shape[0]
    d = table.shape[1]
    mesh = plsc.VectorSubcoreMesh(core_axis_name="c", subcore_axis_name="s")

    @functools.partial(
        pl.kernel,
        out_type=jax.ShapeDtypeStruct((n, d), table.dtype),
        mesh=mesh,
        compiler_params=pltpu.CompilerParams(use_tc_tiling_on_sc=False),
    )
    def gk(table_hbm, ids_hbm, out_hbm):
        def body(i_vmem, o_vmem):
            pltpu.sync_copy(table_hbm.at[i_vmem], o_vmem)

        pltpu.emit_pipeline(
            body,
            grid=(n // GATHER_WINDOW,),
            in_specs=[pl.BlockSpec((GATHER_WINDOW,), lambda i: (i,))],
            out_specs=[pl.BlockSpec((GATHER_WINDOW, d), lambda i: (i, 0))],
            core_axis_name=("c", "s"),
            dimension_semantics=(pltpu.PARALLEL,),
        )(ids_hbm, out_hbm)

    return gk(table, ids1)


def _mlp_concat(xr, W1, b1, W2, b2, bsz, block_b=512):
    k = W1.shape[0]
    hid = W1.shape[1]
    out = W2.shape[1]
    n_chunks = k // 128
    n_b = bsz // block_b

    def body(*refs):
        xs = refs[:n_chunks]
        w1_ref, b1_ref, w2_ref, b2_ref, o_ref = refs[n_chunks:]
        x = jnp.concatenate([r[...] for r in xs], axis=1).astype(jnp.bfloat16)
        h = jnp.dot(x, w1_ref[...].astype(jnp.bfloat16),
                    preferred_element_type=jnp.float32) + b1_ref[...]
        h = jnp.maximum(h, 0.0).astype(jnp.bfloat16)
        o = jnp.dot(h, w2_ref[...].astype(jnp.bfloat16),
                    preferred_element_type=jnp.float32) + b2_ref[...]
        o_ref[...] = jnp.maximum(o, 0.0)

    x_specs = [
        pl.BlockSpec((block_b, 128), lambda i, J=j: (J * n_b + i, 0))
        for j in range(n_chunks)
    ]
    return pl.pallas_call(
        body,
        grid=(n_b,),
        in_specs=x_specs + [
            pl.BlockSpec((k, hid), lambda i: (0, 0)),
            pl.BlockSpec((1, hid), lambda i: (0, 0)),
            pl.BlockSpec((hid, out), lambda i: (0, 0)),
            pl.BlockSpec((1, out), lambda i: (0, 0)),
        ],
        out_specs=pl.BlockSpec((block_b, out), lambda i: (i, 0)),
        out_shape=jax.ShapeDtypeStruct((bsz, out), jnp.float32),
        compiler_params=pltpu.CompilerParams(
            dimension_semantics=("parallel",)),
    )(*([xr] * n_chunks), W1, b1, W2, b2)


def kernel(input_ids, emb, W1, b1, W2, b2):
    bsz, seq = input_ids.shape
    d = emb.shape[1]
    emb128 = lax.optimization_barrier(
        jnp.concatenate([emb[0::2], emb[1::2]], axis=1))
    table = emb128.reshape(emb.shape)
    n_chunks = seq * d // 128
    ids_perm = (input_ids.astype(jnp.int32)
                .reshape(bsz, n_chunks, 2)
                .transpose(1, 0, 2)
                .reshape(-1))
    g = _sc_gather(table, ids_perm)
    xr = g.reshape(-1, 2 * d)
    return _mlp_concat(xr, W1, b1.reshape(1, -1), W2, b2.reshape(1, -1), bsz)


# speedup vs baseline: 5.0844x; 5.0844x over previous
import functools

import jax
import jax.numpy as jnp
from jax import lax
from jax.experimental import pallas as pl
from jax.experimental.pallas import tpu as pltpu
from jax.experimental.pallas import tpu_sc as plsc

BW = 128


GATHER_WINDOW = 512


def _sc_gather(table, ids1):
    n = ids1.shape[0]
    d = table.shape[1]
    mesh = plsc.VectorSubcoreMesh(core_axis_name="c", subcore_axis_name="s")

    @functools.partial(
        pl.kernel,
        out_type=jax.ShapeDtypeStruct((n, d), table.dtype),
        mesh=mesh,
        compiler_params=pltpu.CompilerParams(use_tc_tiling_on_sc=False),
    )
    def gk(table_hbm, ids_hbm, out_hbm):
        def body(i_vmem, o_vmem):
            pltpu.sync_copy(table_hbm.at[i_vmem], o_vmem)

        pltpu.emit_pipeline(
            body,
            grid=(n // GATHER_WINDOW,),
            in_specs=[pl.BlockSpec((GATHER_WINDOW,), lambda i: (i,))],
            out_specs=[pl.BlockSpec((GATHER_WINDOW, d), lambda i: (i, 0))],
            core_axis_name=("c", "s"),
            dimension_semantics=(pltpu.PARALLEL,),
        )(ids_hbm, out_hbm)

    return gk(table, ids1)


def _mlp_concat(xr, W1, b1, W2, b2, bsz, block_b=512):
    k = W1.shape[0]
    hid = W1.shape[1]
    out = W2.shape[1]
    n_chunks = k // 128
    n_b = bsz // block_b

    def body(*refs):
        xs = refs[:n_chunks]
        w1_ref, b1_ref, w2_ref, b2_ref, o_ref = refs[n_chunks:]
        x = jnp.concatenate([r[...] for r in xs], axis=1).astype(jnp.bfloat16)
        h = jnp.dot(x, w1_ref[...].astype(jnp.bfloat16),
                    preferred_element_type=jnp.float32) + b1_ref[...]
        h = jnp.maximum(h, 0.0).astype(jnp.bfloat16)
        o = jnp.dot(h, w2_ref[...].astype(jnp.bfloat16),
                    preferred_element_type=jnp.float32) + b2_ref[...]
        o_ref[...] = jnp.maximum(o, 0.0)

    x_specs = [
        pl.BlockSpec((block_b, 128), lambda i, J=j: (J * n_b + i, 0))
        for j in range(n_chunks)
    ]
    return pl.pallas_call(
        body,
        grid=(n_b,),
        in_specs=x_specs + [
            pl.BlockSpec((k, hid), lambda i: (0, 0)),
            pl.BlockSpec((1, hid), lambda i: (0, 0)),
            pl.BlockSpec((hid, out), lambda i: (0, 0)),
            pl.BlockSpec((1, out), lambda i: (0, 0)),
        ],
        out_specs=pl.BlockSpec((block_b, out), lambda i: (i, 0)),
        out_shape=jax.ShapeDtypeStruct((bsz, out), jnp.float32),
        compiler_params=pltpu.CompilerParams(
            dimension_semantics=("parallel",)),
    )(*([xr] * n_chunks), W1, b1, W2, b2)


def kernel(input_ids, emb, W1, b1, W2, b2):
    bsz, seq = input_ids.shape
    d = emb.shape[1]
    emb128 = lax.optimization_barrier(emb.reshape(-1, 2 * d))
    table = emb128.reshape(emb.shape)
    n_chunks = seq * d // 128
    ids_perm = (input_ids.astype(jnp.int32)
                .reshape(bsz, n_chunks, 2)
                .transpose(1, 0, 2)
                .reshape(-1))
    g = _sc_gather(table, ids_perm)
    xr = g.reshape(-1, 2 * d)
    return _mlp_concat(xr, W1, b1.reshape(1, -1), W2, b2.reshape(1, -1), bsz)


# speedup vs baseline: 6.5429x; 1.2869x over previous
import functools

import jax
import jax.numpy as jnp
from jax import lax
from jax.experimental import pallas as pl
from jax.experimental.pallas import tpu as pltpu
from jax.experimental.pallas import tpu_sc as plsc

BW = 128


GATHER_WINDOW = 512


def _sc_gather(table, ids1):
    n = ids1.shape[0]
    d = table.shape[1]
    mesh = plsc.VectorSubcoreMesh(core_axis_name="c", subcore_axis_name="s")

    @functools.partial(
        pl.kernel,
        out_type=jax.ShapeDtypeStruct((n, d), table.dtype),
        mesh=mesh,
        compiler_params=pltpu.CompilerParams(use_tc_tiling_on_sc=False),
    )
    def gk(table_hbm, ids_hbm, out_hbm):
        def body(i_vmem, o_vmem):
            pltpu.sync_copy(table_hbm.at[i_vmem], o_vmem)

        pltpu.emit_pipeline(
            body,
            grid=(n // GATHER_WINDOW,),
            in_specs=[pl.BlockSpec((GATHER_WINDOW,), lambda i: (i,))],
            out_specs=[pl.BlockSpec((GATHER_WINDOW, d), lambda i: (i, 0))],
            core_axis_name=("c", "s"),
            dimension_semantics=(pltpu.PARALLEL,),
        )(ids_hbm, out_hbm)

    return gk(table, ids1)


def _mlp_concat(xr, W1, b1, W2, b2, bsz, block_b=512):
    k = W1.shape[0]
    hid = W1.shape[1]
    out = W2.shape[1]
    n_chunks = k // 128
    n_b = bsz // block_b

    def body(*refs):
        xs = refs[:n_chunks]
        w1_ref, b1_ref, w2_ref, b2_ref, o_ref = refs[n_chunks:]
        x = jnp.concatenate([r[...] for r in xs], axis=1).astype(jnp.bfloat16)
        h = jnp.dot(x, w1_ref[...].astype(jnp.bfloat16),
                    preferred_element_type=jnp.float32) + b1_ref[...]
        h = jnp.maximum(h, 0.0).astype(jnp.bfloat16)
        o = jnp.dot(h, w2_ref[...].astype(jnp.bfloat16),
                    preferred_element_type=jnp.float32) + b2_ref[...]
        o_ref[...] = jnp.maximum(o, 0.0)

    x_specs = [
        pl.BlockSpec((block_b, 128), lambda i, J=j: (J * n_b + i, 0))
        for j in range(n_chunks)
    ]
    return pl.pallas_call(
        body,
        grid=(n_b,),
        in_specs=x_specs + [
            pl.BlockSpec((k, hid), lambda i: (0, 0)),
            pl.BlockSpec((1, hid), lambda i: (0, 0)),
            pl.BlockSpec((hid, out), lambda i: (0, 0)),
            pl.BlockSpec((1, out), lambda i: (0, 0)),
        ],
        out_specs=pl.BlockSpec((block_b, out), lambda i: (i, 0)),
        out_shape=jax.ShapeDtypeStruct((bsz, out), jnp.float32),
        compiler_params=pltpu.CompilerParams(
            dimension_semantics=("parallel",)),
    )(*([xr] * n_chunks), W1, b1, W2, b2)


def kernel(input_ids, emb, W1, b1, W2, b2):
    bsz, seq = input_ids.shape
    d = emb.shape[1]
    emb128 = lax.optimization_barrier(emb.reshape(-1, 2 * d))
    table = emb128.reshape(emb.shape)
    idsT = input_ids.astype(jnp.int32).T
    ev = lax.pad(idsT[0::2], 0, [(0, 0, 0), (0, 1, 1)])
    od = lax.pad(idsT[1::2], 0, [(0, 0, 0), (1, 0, 1)])
    ids_perm = (ev + od).reshape(-1)
    g = _sc_gather(table, ids_perm)
    xr = g.reshape(-1, 2 * d)
    return _mlp_concat(xr, W1, b1.reshape(1, -1), W2, b2.reshape(1, -1), bsz)
